# baseline (speedup 1.0000x reference)
import functools

import jax
import jax.numpy as jnp
from jax import lax
from jax.experimental import pallas as pl
from jax.experimental.pallas import tpu as pltpu
from jax.experimental.pallas import tpu_sc as plsc

_NC = 2
_NS = 16
_L = 16
_CHUNK = 128


@functools.cache
def _build(B, D):
    NW = _NC * _NS
    bpw = B // NW
    nch = bpw // _CHUNK
    groups = bpw // _L
    mesh = plsc.VectorSubcoreMesh(core_axis_name="c", subcore_axis_name="s",
                                  num_cores=_NC, num_subcores=_NS)

    @functools.partial(
        pl.kernel,
        out_type=jax.ShapeDtypeStruct((B,), jnp.float32),
        mesh=mesh,
        scratch_types=[
            pltpu.VMEM((nch, _CHUNK), jnp.int32),
            pltpu.VMEM((nch, _CHUNK), jnp.int32),
            pltpu.VMEM((bpw, D), jnp.float32),
            pltpu.VMEM((bpw, D), jnp.float32),
            pltpu.VMEM((bpw,), jnp.float32),
            pltpu.SemaphoreType.DMA,
        ],
        compiler_params=pltpu.CompilerParams(
            needs_layout_passes=False, use_tc_tiling_on_sc=False),
    )
    def sc_kernel(ut, it, uid, iid, out,
                  uidv, iidv, urow, irow, outv, sem):
        wid = lax.axis_index("s") * _NC + lax.axis_index("c")
        base = wid * bpw
        for j in range(nch):
            off = base + j * _CHUNK
            pltpu.sync_copy(uid.at[pl.ds(off, _CHUNK)], uidv.at[j])
            pltpu.sync_copy(iid.at[pl.ds(off, _CHUNK)], iidv.at[j])
        copies = []
        for j in range(nch):
            s = pl.ds(j * _CHUNK, _CHUNK)
            copies.append(pltpu.async_copy(ut.at[uidv.at[j]], urow.at[s], sem))
            copies.append(pltpu.async_copy(it.at[iidv.at[j]], irow.at[s], sem))
        for c in copies:
            c.wait()

        lane = lax.iota(jnp.int32, _L)

        def body(g, carry):
            rows = g * _L + lane
            acc = jnp.zeros((_L,), jnp.float32)
            for d in range(D):
                dv = jnp.full((_L,), d, jnp.int32)
                acc = acc + (plsc.load_gather(urow, [rows, dv]) *
                             plsc.load_gather(irow, [rows, dv]))
            outv[pl.ds(g * _L, _L)] = acc
            return carry

        lax.fori_loop(0, groups, body, 0)
        pltpu.sync_copy(outv, out.at[pl.ds(base, bpw)])

    return sc_kernel


def kernel(user_table, item_table, user_bias_table, item_bias_table,
           user_ids, item_ids):
    B = user_ids.shape[0]
    D = user_table.shape[1]
    f = _build(B, D)
    uid = user_ids.reshape(B)
    iid = item_ids.reshape(B)
    dot = f(user_table, item_table, uid, iid)
    bias = user_bias_table[uid, 0] + item_bias_table[iid, 0]
    return (dot + bias).reshape(B, 1)


# speedup vs baseline: 1.4712x; 1.4712x over previous
import functools

import jax
import jax.numpy as jnp
from jax import lax
from jax.experimental import pallas as pl
from jax.experimental.pallas import tpu as pltpu
from jax.experimental.pallas import tpu_sc as plsc

_NC = 2
_NS = 16
_L = 16
_CHUNK = 128


@functools.cache
def _build(B, D):
    NW = _NC * _NS
    bpw = B // NW
    nch = bpw // _CHUNK
    groups = bpw // _L
    mesh = plsc.VectorSubcoreMesh(core_axis_name="c", subcore_axis_name="s",
                                  num_cores=_NC, num_subcores=_NS)

    @functools.partial(
        pl.kernel,
        out_type=jax.ShapeDtypeStruct((B,), jnp.float32),
        mesh=mesh,
        scratch_types=[
            pltpu.VMEM((nch, _CHUNK), jnp.int32),
            pltpu.VMEM((nch, _CHUNK), jnp.int32),
            pltpu.VMEM((bpw // 2, 32), jnp.float32),
            pltpu.VMEM((bpw // 2, 32), jnp.float32),
            pltpu.VMEM((bpw,), jnp.float32),
            pltpu.SemaphoreType.DMA,
        ],
        compiler_params=pltpu.CompilerParams(
            needs_layout_passes=False, use_tc_tiling_on_sc=True),
    )
    def sc_kernel(ut, it, uid, iid, out,
                  uidv, iidv, urow, irow, outv, sem):
        wid = lax.axis_index("s") * _NC + lax.axis_index("c")
        base = wid * bpw
        for j in range(nch):
            off = base + j * _CHUNK
            pltpu.sync_copy(uid.at[pl.ds(off, _CHUNK)], uidv.at[j])
            pltpu.sync_copy(iid.at[pl.ds(off, _CHUNK)], iidv.at[j])

        hb = bpw // 2
        hgroups = hb // _L
        lane = lax.iota(jnp.int32, _L)

        for h in range(2):
            hoff = h * hgroups

            def issue(g, carry):
                gg = g + hoff
                s = pl.ds((gg & 7) * _L, _L)
                uvec = uidv.at[gg >> 3][s]
                ivec = iidv.at[gg >> 3][s]
                for kk in range(_L):
                    r = pl.ds(g * _L + kk, 1)
                    pltpu.async_copy(ut.at[pl.ds(uvec[kk], 1)],
                                     urow.at[r], sem)
                    pltpu.async_copy(it.at[pl.ds(ivec[kk], 1)],
                                     irow.at[r], sem)
                return carry

            lax.fori_loop(0, hgroups, issue, 0)
            pltpu.make_async_copy(ut.at[pl.ds(0, hb)], urow, sem).wait()
            pltpu.make_async_copy(it.at[pl.ds(0, hb)], irow, sem).wait()

            def body(g, carry):
                rows = g * _L + lane
                acc = jnp.zeros((_L,), jnp.float32)
                for d in range(D):
                    dv = jnp.full((_L,), d, jnp.int32)
                    acc = acc + (plsc.load_gather(urow, [rows, dv]) *
                                 plsc.load_gather(irow, [rows, dv]))
                outv[pl.ds((g + hoff) * _L, _L)] = acc
                return carry

            lax.fori_loop(0, hgroups, body, 0)
        pltpu.sync_copy(outv, out.at[pl.ds(base, bpw)])

    return sc_kernel


def kernel(user_table, item_table, user_bias_table, item_bias_table,
           user_ids, item_ids):
    B = user_ids.shape[0]
    D = user_table.shape[1]
    f = _build(B, D)
    uid = user_ids.reshape(B)
    iid = item_ids.reshape(B)
    dot = f(user_table, item_table, uid, iid)
    bias = user_bias_table[uid, 0] + item_bias_table[iid, 0]
    return (dot + bias).reshape(B, 1)
